# baseline (speedup 1.0000x reference)
import jax
import jax.numpy as jnp
from jax.experimental import pallas as pl
from jax.experimental.pallas import tpu as pltpu
from jax.experimental.pallas import tpu_sc as plsc

_T_END = 4096
_M = 2048


def _row_lookup_body(u_hbm, t_hbm, z_hbm, out_hbm, t_s, row_sp):
    pltpu.sync_copy(t_hbm, t_s)
    t = t_s[0]
    safe_t = jnp.minimum(t, _T_END - 1)
    valid = t < _T_END

    @pl.when(valid)
    def _copy_row():
        pltpu.sync_copy(u_hbm.at[safe_t], row_sp)

    @pl.when(jnp.logical_not(valid))
    def _copy_zeros():
        pltpu.sync_copy(z_hbm, row_sp)

    pltpu.sync_copy(row_sp, out_hbm)


def kernel(u, t):
    t_vec = jnp.reshape(jnp.asarray(t, jnp.int32), (1,))
    zeros_row = jnp.zeros((_M,), jnp.float32)
    f = pl.kernel(
        _row_lookup_body,
        out_type=jax.ShapeDtypeStruct((_M,), jnp.float32),
        mesh=plsc.ScalarSubcoreMesh(axis_name="c", num_cores=1),
        scratch_types=[
            pltpu.SMEM((1,), jnp.int32),
            pltpu.VMEM_SHARED((_M,), jnp.float32),
        ],
    )
    return f(u, t_vec, zeros_row)
